# speedup vs baseline: 1.0287x; 1.0287x over previous
import functools

import jax
import jax.numpy as jnp
from jax.experimental import pallas as pl
from jax.experimental.pallas import tpu as pltpu


def _gru2_fc_kernel(x_ref,
                    wih0_ref, whh0_ref, bx0_ref, bhn0_ref,
                    wih1_ref, whh1_ref, bx1_ref, bhn1_ref,
                    fcw_ref, fcb_ref,
                    out_ref, gx_scr, seq_scr, *, T, Bc, H):
    f32 = jnp.float32
    D = x_ref.shape[-1]

    xin = x_ref[...].reshape(T * Bc, D)
    gx_scr[...] = (jnp.dot(xin, wih0_ref[...], preferred_element_type=f32)
                   + bx0_ref[...])

    def gru_step(t, h, whh, bhn, write_seq):
        row = pl.multiple_of(t * Bc, Bc)
        gh = jnp.dot(h.astype(whh.dtype), whh,
                     preferred_element_type=f32) + bhn
        g = gx_scr[pl.ds(row, Bc), :]
        r = jax.nn.sigmoid(g[:, 0 * H:1 * H] + gh[:, 0 * H:1 * H])
        z = jax.nn.sigmoid(g[:, 1 * H:2 * H] + gh[:, 1 * H:2 * H])
        n = jnp.tanh(g[:, 2 * H:3 * H] + r * gh[:, 2 * H:3 * H])
        h_new = n + z * (h - n)
        if write_seq:
            seq_scr[pl.ds(row, Bc), :] = h_new.astype(seq_scr.dtype)
        return h_new

    whh0 = whh0_ref[...]
    bhn0 = bhn0_ref[...]
    jax.lax.fori_loop(
        0, T,
        lambda t, h: gru_step(t, h, whh0, bhn0, True),
        jnp.zeros((Bc, H), f32), unroll=True)

    gx_scr[...] = (jnp.dot(seq_scr[...], wih1_ref[...],
                           preferred_element_type=f32) + bx1_ref[...])

    whh1 = whh1_ref[...]
    bhn1 = bhn1_ref[...]
    h_last = jax.lax.fori_loop(
        0, T,
        lambda t, h: gru_step(t, h, whh1, bhn1, False),
        jnp.zeros((Bc, H), f32), unroll=True)

    out_ref[...] = (jnp.dot(h_last.astype(fcw_ref.dtype), fcw_ref[...],
                            preferred_element_type=f32)
                    + fcb_ref[...]).astype(out_ref.dtype)


def _prep_layer(w_ih, w_hh, b_ih, b_hh, H):
    bf16 = jnp.bfloat16
    wih_t = w_ih.T.astype(bf16)
    whh_t = w_hh.T.astype(bf16)
    zeros_h = jnp.zeros((H,), jnp.float32)
    bx = (b_ih + jnp.concatenate([b_hh[:2 * H], zeros_h])).reshape(1, 3 * H)
    bhn = jnp.concatenate([zeros_h, zeros_h, b_hh[2 * H:]]).reshape(1, 3 * H)
    return wih_t, whh_t, bx, bhn


def kernel(w_ih_0, w_hh_0, b_ih_0, b_hh_0,
           w_ih_1, w_hh_1, b_ih_1, b_hh_1,
           fc_w, fc_b, x):
    B, T, D = x.shape
    H = w_hh_0.shape[1]
    C = fc_w.shape[0]
    bf16 = jnp.bfloat16

    n_shards = 2
    Bc = B // n_shards

    wih0, whh0, bx0, bhn0 = _prep_layer(w_ih_0, w_hh_0, b_ih_0, b_hh_0, H)
    wih1, whh1, bx1, bhn1 = _prep_layer(w_ih_1, w_hh_1, b_ih_1, b_hh_1, H)
    fcw = fc_w.T.astype(bf16)
    fcb = fc_b.reshape(1, C)

    x_tm = jnp.transpose(x, (1, 0, 2)).astype(bf16)

    operands = [x_tm, wih0, whh0, bx0, bhn0, wih1, whh1, bx1, bhn1, fcw, fcb]
    in_specs = [pl.BlockSpec((T, Bc, D), lambda i: (0, i, 0))]
    for a in operands[1:]:
        in_specs.append(pl.BlockSpec(a.shape, lambda i, nd=a.ndim: (0,) * nd))

    out = pl.pallas_call(
        functools.partial(_gru2_fc_kernel, T=T, Bc=Bc, H=H),
        out_shape=jax.ShapeDtypeStruct((B, C), jnp.float32),
        grid=(n_shards,),
        in_specs=in_specs,
        out_specs=pl.BlockSpec((Bc, C), lambda i: (i, 0)),
        scratch_shapes=[
            pltpu.VMEM((T * Bc, 3 * H), jnp.float32),
            pltpu.VMEM((T * Bc, H), bf16),
        ],
        compiler_params=pltpu.CompilerParams(
            dimension_semantics=("parallel",)),
    )(*operands)
    return out


# speedup vs baseline: 1.2411x; 1.2065x over previous
import functools

import jax
import jax.numpy as jnp
from jax.experimental import pallas as pl
from jax.experimental.pallas import tpu as pltpu


def _gru2_fc_kernel(x_ref,
                    wih0_ref, whh0_ref, bx0_ref, bhn0_ref,
                    wih1_ref, whh1_ref, bx1_ref, bhn1_ref,
                    fcw_ref, fcb_ref,
                    out_ref, gx_scr, seq_scr, *, T, B, H):
    f32 = jnp.float32

    gx_scr[...] = (jnp.dot(x_ref[...], wih0_ref[...],
                           preferred_element_type=f32) + bx0_ref[...])

    def gru_step(t, h, whh, bhn, write_seq):
        row = pl.multiple_of(t * B, B)
        gh = jnp.dot(h.astype(whh.dtype), whh, preferred_element_type=f32)
        g = gx_scr[pl.ds(row, B), :]
        tr = jnp.tanh(g[:, 0 * H:1 * H] + gh[:, 0 * H:1 * H])
        tz = jnp.tanh(g[:, 1 * H:2 * H] + gh[:, 1 * H:2 * H])
        ghn = gh[:, 2 * H:3 * H] + bhn
        n = jnp.tanh(g[:, 2 * H:3 * H] + ghn + tr * ghn)
        h_new = 0.5 * ((h + n) + tz * (h - n))
        if write_seq:
            seq_scr[pl.ds(row, B), :] = h_new.astype(seq_scr.dtype)
        return h_new

    whh0 = whh0_ref[...]
    bhn0 = bhn0_ref[...]
    jax.lax.fori_loop(
        0, T,
        lambda t, h: gru_step(t, h, whh0, bhn0, True),
        jnp.zeros((B, H), f32), unroll=True)

    gx_scr[...] = (jnp.dot(seq_scr[...], wih1_ref[...],
                           preferred_element_type=f32) + bx1_ref[...])

    whh1 = whh1_ref[...]
    bhn1 = bhn1_ref[...]
    h_last = jax.lax.fori_loop(
        0, T,
        lambda t, h: gru_step(t, h, whh1, bhn1, False),
        jnp.zeros((B, H), f32), unroll=True)

    out_ref[...] = (jnp.dot(h_last.astype(fcw_ref.dtype), fcw_ref[...],
                            preferred_element_type=f32)
                    + fcb_ref[...]).astype(out_ref.dtype)


def _prep_layer(w_ih, w_hh, b_ih, b_hh, H):
    bf16 = jnp.bfloat16
    scale = jnp.concatenate([jnp.full((2 * H,), 0.5, jnp.float32),
                             jnp.ones((H,), jnp.float32)])
    wih_t = (w_ih.T * scale[None, :]).astype(bf16)
    whh_t = (0.5 * w_hh.T).astype(bf16)
    bx = (scale * (b_ih + jnp.concatenate(
        [b_hh[:2 * H], jnp.zeros((H,), jnp.float32)]))).reshape(1, 3 * H)
    bhn = (0.5 * b_hh[2 * H:]).reshape(1, H)
    return wih_t, whh_t, bx, bhn


def kernel(w_ih_0, w_hh_0, b_ih_0, b_hh_0,
           w_ih_1, w_hh_1, b_ih_1, b_hh_1,
           fc_w, fc_b, x):
    B, T, D = x.shape
    H = w_hh_0.shape[1]
    C = fc_w.shape[0]
    bf16 = jnp.bfloat16

    wih0, whh0, bx0, bhn0 = _prep_layer(w_ih_0, w_hh_0, b_ih_0, b_hh_0, H)
    wih1, whh1, bx1, bhn1 = _prep_layer(w_ih_1, w_hh_1, b_ih_1, b_hh_1, H)
    fcw = fc_w.T.astype(bf16)
    fcb = fc_b.reshape(1, C)

    x_flat = jnp.transpose(x, (1, 0, 2)).astype(bf16).reshape(T * B, D)

    operands = [x_flat, wih0, whh0, bx0, bhn0, wih1, whh1, bx1, bhn1,
                fcw, fcb]
    in_specs = [pl.BlockSpec(a.shape, lambda i, nd=a.ndim: (0,) * nd)
                for a in operands]

    out = pl.pallas_call(
        functools.partial(_gru2_fc_kernel, T=T, B=B, H=H),
        out_shape=jax.ShapeDtypeStruct((B, C), jnp.float32),
        grid=(1,),
        in_specs=in_specs,
        out_specs=pl.BlockSpec((B, C), lambda i: (0, 0)),
        scratch_shapes=[
            pltpu.VMEM((T * B, 3 * H), jnp.float32),
            pltpu.VMEM((T * B, H), bf16),
        ],
        compiler_params=pltpu.CompilerParams(
            dimension_semantics=("arbitrary",)),
    )(*operands)
    return out


# speedup vs baseline: 1.2919x; 1.0410x over previous
import functools

import jax
import jax.numpy as jnp
from jax.experimental import pallas as pl
from jax.experimental.pallas import tpu as pltpu


def _gru2_fc_kernel(x_ref,
                    wih0_ref, whh0_ref, bx0_ref, bhn0_ref,
                    wih1_ref, whh1_ref, bx1_ref, bhn1_ref,
                    fcw_ref, fcb_ref,
                    out_ref, gx_scr, seq_scr, *, T, B, H):
    f32 = jnp.float32
    bf16 = jnp.bfloat16

    gx_scr[...] = (jnp.dot(x_ref[...], wih0_ref[...],
                           preferred_element_type=f32)
                   + bx0_ref[...]).astype(bf16)

    def gru_step(t, h, whh, bhn, write_seq):
        row = pl.multiple_of(t * B, B)
        gh = jnp.dot(h, whh, preferred_element_type=f32).astype(bf16)
        g = gx_scr[pl.ds(row, B), :]
        tr = jnp.tanh(g[:, 0 * H:1 * H] + gh[:, 0 * H:1 * H])
        tz = jnp.tanh(g[:, 1 * H:2 * H] + gh[:, 1 * H:2 * H])
        ghn = gh[:, 2 * H:3 * H] + bhn
        n = jnp.tanh(g[:, 2 * H:3 * H] + ghn + tr * ghn)
        h_new = bf16(0.5) * ((h + n) + tz * (h - n))
        if write_seq:
            seq_scr[pl.ds(row, B), :] = h_new
        return h_new

    whh0 = whh0_ref[...]
    bhn0 = bhn0_ref[...]
    jax.lax.fori_loop(
        0, T,
        lambda t, h: gru_step(t, h, whh0, bhn0, True),
        jnp.zeros((B, H), bf16), unroll=True)

    gx_scr[...] = (jnp.dot(seq_scr[...], wih1_ref[...],
                           preferred_element_type=f32)
                   + bx1_ref[...]).astype(bf16)

    whh1 = whh1_ref[...]
    bhn1 = bhn1_ref[...]
    h_last = jax.lax.fori_loop(
        0, T,
        lambda t, h: gru_step(t, h, whh1, bhn1, False),
        jnp.zeros((B, H), bf16), unroll=True)

    out_ref[...] = (jnp.dot(h_last, fcw_ref[...],
                            preferred_element_type=f32)
                    + fcb_ref[...]).astype(out_ref.dtype)


def _prep_layer(w_ih, w_hh, b_ih, b_hh, H):
    bf16 = jnp.bfloat16
    scale = jnp.concatenate([jnp.full((2 * H,), 0.5, jnp.float32),
                             jnp.ones((H,), jnp.float32)])
    wih_t = (w_ih.T * scale[None, :]).astype(bf16)
    whh_t = (0.5 * w_hh.T).astype(bf16)
    bx = (scale * (b_ih + jnp.concatenate(
        [b_hh[:2 * H], jnp.zeros((H,), jnp.float32)]))).reshape(1, 3 * H)
    bhn = (0.5 * b_hh[2 * H:]).reshape(1, H).astype(bf16)
    return wih_t, whh_t, bx, bhn


def kernel(w_ih_0, w_hh_0, b_ih_0, b_hh_0,
           w_ih_1, w_hh_1, b_ih_1, b_hh_1,
           fc_w, fc_b, x):
    B, T, D = x.shape
    H = w_hh_0.shape[1]
    C = fc_w.shape[0]
    bf16 = jnp.bfloat16

    wih0, whh0, bx0, bhn0 = _prep_layer(w_ih_0, w_hh_0, b_ih_0, b_hh_0, H)
    wih1, whh1, bx1, bhn1 = _prep_layer(w_ih_1, w_hh_1, b_ih_1, b_hh_1, H)
    fcw = fc_w.T.astype(bf16)
    fcb = fc_b.reshape(1, C)

    x_flat = jnp.transpose(x, (1, 0, 2)).astype(bf16).reshape(T * B, D)

    operands = [x_flat, wih0, whh0, bx0, bhn0, wih1, whh1, bx1, bhn1,
                fcw, fcb]
    in_specs = [pl.BlockSpec(a.shape, lambda i, nd=a.ndim: (0,) * nd)
                for a in operands]

    out = pl.pallas_call(
        functools.partial(_gru2_fc_kernel, T=T, B=B, H=H),
        out_shape=jax.ShapeDtypeStruct((B, C), jnp.float32),
        grid=(1,),
        in_specs=in_specs,
        out_specs=pl.BlockSpec((B, C), lambda i: (0, 0)),
        scratch_shapes=[
            pltpu.VMEM((T * B, 3 * H), bf16),
            pltpu.VMEM((T * B, H), bf16),
        ],
        compiler_params=pltpu.CompilerParams(
            dimension_semantics=("arbitrary",)),
    )(*operands)
    return out


# speedup vs baseline: 1.5429x; 1.1943x over previous
import functools

import jax
import jax.numpy as jnp
from jax.experimental import pallas as pl
from jax.experimental.pallas import tpu as pltpu


def _gru2_fc_kernel(x_ref,
                    wih0_ref, whh0_ref, bx0_ref, bhn0_ref,
                    wih1_ref, whh1_ref, bx1_ref, bhn1_ref,
                    fcw_ref, fcb_ref,
                    out_ref, gx_scr, *, T, B, H):
    f32 = jnp.float32
    bf16 = jnp.bfloat16
    half = bf16(0.5)

    gx_scr[...] = jnp.dot(x_ref[...], wih0_ref[...],
                          preferred_element_type=f32).astype(bf16)

    whh0 = whh0_ref[...]
    whh1 = whh1_ref[...]
    wih1 = wih1_ref[...]
    bx0 = bx0_ref[...].astype(bf16)
    bhn0 = bhn0_ref[...].astype(bf16)
    bhn1 = bhn1_ref[...].astype(bf16)
    bx1 = bx1_ref[...]

    def gru_step(g, h, whh, bhn):
        gh = jnp.dot(h, whh, preferred_element_type=f32).astype(bf16)
        tr = jnp.tanh(g[:, 0 * H:1 * H] + gh[:, 0 * H:1 * H])
        tz = jnp.tanh(g[:, 1 * H:2 * H] + gh[:, 1 * H:2 * H])
        ghn = gh[:, 2 * H:3 * H] + bhn
        n = jnp.tanh(g[:, 2 * H:3 * H] + ghn + tr * ghn)
        return half * ((h + n) + tz * (h - n))

    h0 = jnp.zeros((B, H), bf16)
    h1 = jnp.zeros((B, H), bf16)
    g1 = None
    for t in range(T):
        row = pl.multiple_of(t * B, B)
        g0 = gx_scr[pl.ds(row, B), :] + bx0
        h0 = gru_step(g0, h0, whh0, bhn0)
        if t >= 1:
            h1 = gru_step(g1, h1, whh1, bhn1)
        g1 = (jnp.dot(h0, wih1, preferred_element_type=f32)
              + bx1).astype(bf16)
    h1 = gru_step(g1, h1, whh1, bhn1)

    out_ref[...] = (jnp.dot(h1, fcw_ref[...], preferred_element_type=f32)
                    + fcb_ref[...]).astype(out_ref.dtype)


def _prep_layer(w_ih, w_hh, b_ih, b_hh, H):
    bf16 = jnp.bfloat16
    scale = jnp.concatenate([jnp.full((2 * H,), 0.5, jnp.float32),
                             jnp.ones((H,), jnp.float32)])
    wih_t = (w_ih.T * scale[None, :]).astype(bf16)
    whh_t = (0.5 * w_hh.T).astype(bf16)
    bx = (scale * (b_ih + jnp.concatenate(
        [b_hh[:2 * H], jnp.zeros((H,), jnp.float32)]))).reshape(1, 3 * H)
    bhn = (0.5 * b_hh[2 * H:]).reshape(1, H)
    return wih_t, whh_t, bx, bhn


def kernel(w_ih_0, w_hh_0, b_ih_0, b_hh_0,
           w_ih_1, w_hh_1, b_ih_1, b_hh_1,
           fc_w, fc_b, x):
    B, T, D = x.shape
    H = w_hh_0.shape[1]
    C = fc_w.shape[0]
    bf16 = jnp.bfloat16

    wih0, whh0, bx0, bhn0 = _prep_layer(w_ih_0, w_hh_0, b_ih_0, b_hh_0, H)
    wih1, whh1, bx1, bhn1 = _prep_layer(w_ih_1, w_hh_1, b_ih_1, b_hh_1, H)
    fcw = fc_w.T.astype(bf16)
    fcb = fc_b.reshape(1, C)

    x_flat = jnp.transpose(x, (1, 0, 2)).astype(bf16).reshape(T * B, D)

    operands = [x_flat, wih0, whh0, bx0, bhn0, wih1, whh1, bx1, bhn1,
                fcw, fcb]
    in_specs = [pl.BlockSpec(a.shape, lambda i, nd=a.ndim: (0,) * nd)
                for a in operands]

    out = pl.pallas_call(
        functools.partial(_gru2_fc_kernel, T=T, B=B, H=H),
        out_shape=jax.ShapeDtypeStruct((B, C), jnp.float32),
        grid=(1,),
        in_specs=in_specs,
        out_specs=pl.BlockSpec((B, C), lambda i: (0, 0)),
        scratch_shapes=[
            pltpu.VMEM((T * B, 3 * H), bf16),
        ],
        compiler_params=pltpu.CompilerParams(
            dimension_semantics=("arbitrary",)),
    )(*operands)
    return out
